# baseline (speedup 1.0000x reference)
import functools

import jax
import jax.numpy as jnp
from jax import lax
from jax.experimental import pallas as pl
from jax.experimental.pallas import tpu as pltpu
from jax.experimental.pallas import tpu_sc as plsc

_BATCH = 4096
_SEQ = 200
_D = 128
_B = _BATCH * _SEQ
_NW = 32
_B_PER_W = _B // _NW
_CHUNK = 128
_N_CHUNKS = _B_PER_W // _CHUNK
_PAIR = 2 * _CHUNK
_N_PAIRS = _N_CHUNKS // 2
_NBUF = 2
_V = 1000


def _emb_body(idx_hbm, table_hbm, out_hbm, idx_v, rows, table_sp, gsems,
              ssems):
    sid = lax.axis_index("s")
    wid = sid * 2 + lax.axis_index("c")
    base = wid * _B_PER_W
    @pl.when(sid == 0)
    def _():
        pltpu.sync_copy(table_hbm, table_sp)
    pltpu.sync_copy(idx_hbm.at[wid], idx_v)
    plsc.subcore_barrier()

    def gather_pair(p, b):
        pltpu.async_copy(table_sp.at[idx_v.at[2 * p]],
                         rows[b].at[pl.ds(0, _CHUNK)], gsems[b])
        pltpu.async_copy(table_sp.at[idx_v.at[2 * p + 1]],
                         rows[b].at[pl.ds(_CHUNK, _CHUNK)], gsems[b])

    def gwait(b):
        for _ in range(2):
            pltpu.make_async_copy(table_sp.at[idx_v.at[0]],
                                  rows[b].at[pl.ds(0, _CHUNK)],
                                  gsems[b]).wait()

    def store(p, b):
        pltpu.async_copy(rows[b],
                         out_hbm.at[pl.ds(base + p * _PAIR, _PAIR)],
                         ssems[b])

    def swait(b):
        pltpu.make_async_copy(rows[b],
                              out_hbm.at[pl.ds(base, _PAIR)],
                              ssems[b]).wait()

    gather_pair(0, 0)
    gather_pair(1, 1)
    gwait(0)
    store(0, 0)

    def body_pair(t, _):
        swait(0)
        gather_pair(2 * t + 2, 0)
        gwait(1)
        store(2 * t + 1, 1)
        swait(1)
        gather_pair(2 * t + 3, 1)
        gwait(0)
        store(2 * t + 2, 0)
        return 0

    lax.fori_loop(0, (_N_PAIRS - 2) // 2 - 1, body_pair, 0)

    swait(0)
    gather_pair(_N_PAIRS - 2, 0)
    gwait(1)
    store(_N_PAIRS - 3, 1)
    swait(1)
    gather_pair(_N_PAIRS - 1, 1)
    gwait(0)
    store(_N_PAIRS - 2, 0)
    gwait(1)
    store(_N_PAIRS - 1, 1)
    swait(0)
    swait(1)


_emb_kernel = functools.partial(
    pl.kernel,
    out_type=jax.ShapeDtypeStruct((_B, _D), jnp.float32),
    mesh=plsc.VectorSubcoreMesh(core_axis_name="c", subcore_axis_name="s"),
    scratch_types=[
        pltpu.VMEM((_N_CHUNKS, _CHUNK), jnp.int32),
        [pltpu.VMEM((_PAIR, _D), jnp.float32)] * _NBUF,
        pltpu.VMEM_SHARED((_V, _D), jnp.float32),
        [pltpu.SemaphoreType.DMA] * _NBUF,
        [pltpu.SemaphoreType.DMA] * _NBUF,
    ],
)(_emb_body)


def kernel(actions, emb_weight):
    idx = actions.reshape(_NW, _N_CHUNKS, _CHUNK).astype(jnp.int32)
    out = _emb_kernel(idx, emb_weight)
    return out.reshape(_BATCH, _SEQ, _D)
